# Initial kernel scaffold; baseline (speedup 1.0000x reference)
import jax
import jax.numpy as jnp
from jax.experimental import pallas as pl


def kernel(x, edge_index, batch, W1s, b1s, W2s, b2s, gammas, betas, eps_vec, lin1_W, lin1_b, lin2_W, lin2_b):
    raise NotImplementedError("write your pallas kernel here")


# speedup vs baseline: 4.5497x; 4.5497x over previous
import functools

import jax
import jax.numpy as jnp
from jax import lax
from jax.experimental import pallas as pl
from jax.experimental.pallas import tpu as pltpu
from jax.experimental.pallas import tpu_sc as plsc

_N = 10000
_E = 320000
_H = 128
_G = 64
_NC = 2
_NS = 16
_NW = _NC * _NS
_EPW = _E // _NW
_CH = 80
_NCHUNK = _EPW // _CH
_RPT = 624
_TAIL = _N - _NS * _RPT
_RB = 1000
_NRB = _N // _RB


@functools.cache
def _make_sc_segsum():
    mesh = plsc.VectorSubcoreMesh(core_axis_name="c", subcore_axis_name="s")
    out_t = (jax.ShapeDtypeStruct((_N, _H), jnp.float32),
             jax.ShapeDtypeStruct((_N, _H), jnp.float32))

    @functools.partial(
        pl.kernel, mesh=mesh, out_type=out_t,
        scratch_types=[
            pltpu.VMEM((_CH,), jnp.int32),
            pltpu.VMEM((_CH,), jnp.int32),
            pltpu.VMEM((_CH, _H), jnp.float32),
            pltpu.VMEM_SHARED((_N, _H), jnp.float32),
            pltpu.SemaphoreType.DMA,
        ])
    def seg(h_hbm, src_hbm, dst_hbm, out0, out1, src_v, dst_v, rows_v, acc, sem):
        cid = lax.axis_index("c")
        sid = lax.axis_index("s")
        wid = cid * _NS + sid

        def zrow(i, c):
            for j in range(_H // 16):
                rows_v[i, pl.ds(j * 16, 16)] = jnp.zeros((16,), jnp.float32)
            return c
        lax.fori_loop(0, _CH, zrow, 0)
        base_r = sid * _RPT

        def zacc(j, c):
            pltpu.sync_copy(rows_v, acc.at[pl.ds(base_r + j * _CH, _CH)])
            return c
        lax.fori_loop(0, _RPT // _CH, zacc, 0)
        rem = _RPT - (_RPT // _CH) * _CH
        if rem:
            pltpu.sync_copy(rows_v.at[pl.ds(0, rem)],
                            acc.at[pl.ds(base_r + (_RPT // _CH) * _CH, rem)])

        @pl.when(sid == _NS - 1)
        def _():
            pltpu.sync_copy(rows_v.at[pl.ds(0, _TAIL)],
                            acc.at[pl.ds(_NS * _RPT, _TAIL)])
        plsc.subcore_barrier()

        def step(i, c):
            e0 = wid * _EPW + i * _CH
            pltpu.sync_copy(src_hbm.at[pl.ds(e0, _CH)], src_v)
            pltpu.sync_copy(dst_hbm.at[pl.ds(e0, _CH)], dst_v)
            pltpu.async_copy(h_hbm.at[src_v], rows_v, sem).wait()
            pltpu.sync_copy(rows_v, acc.at[dst_v], add=True)
            return c
        lax.fori_loop(0, _NCHUNK, step, 0)
        plsc.subcore_barrier()

        @pl.when(cid == 0)
        def _():
            pltpu.sync_copy(acc.at[pl.ds(base_r, _RPT)], out0.at[pl.ds(base_r, _RPT)])

            @pl.when(sid == _NS - 1)
            def _():
                pltpu.sync_copy(acc.at[pl.ds(_NS * _RPT, _TAIL)],
                                out0.at[pl.ds(_NS * _RPT, _TAIL)])

        @pl.when(cid == 1)
        def _():
            pltpu.sync_copy(acc.at[pl.ds(base_r, _RPT)], out1.at[pl.ds(base_r, _RPT)])

            @pl.when(sid == _NS - 1)
            def _():
                pltpu.sync_copy(acc.at[pl.ds(_NS * _RPT, _TAIL)],
                                out1.at[pl.ds(_NS * _RPT, _TAIL)])

    return seg


def _mlp_body(h_ref, p0_ref, p1_ref, sc_ref, w1_ref, b1_ref, w2_ref, b2_ref,
              z2_ref, st_ref):
    z = h_ref[...] * sc_ref[...] + (p0_ref[...] + p1_ref[...])
    z1 = jnp.maximum(jnp.dot(z, w1_ref[...], preferred_element_type=jnp.float32)
                     + b1_ref[...], 0.0)
    z2 = jnp.maximum(jnp.dot(z1, w2_ref[...], preferred_element_type=jnp.float32)
                     + b2_ref[...], 0.0)
    z2_ref[...] = z2
    s = jnp.sum(z2, axis=0, keepdims=True)
    ss = jnp.sum(z2 * z2, axis=0, keepdims=True)
    upd = jnp.concatenate([s, ss, jnp.zeros((6, _H), jnp.float32)], axis=0)

    @pl.when(pl.program_id(0) == 0)
    def _():
        st_ref[...] = jnp.zeros_like(st_ref)

    st_ref[...] += upd


_mlp_call = pl.pallas_call(
    _mlp_body,
    grid=(_NRB,),
    in_specs=[
        pl.BlockSpec((_RB, _H), lambda i: (i, 0)),
        pl.BlockSpec((_RB, _H), lambda i: (i, 0)),
        pl.BlockSpec((_RB, _H), lambda i: (i, 0)),
        pl.BlockSpec((1, _H), lambda i: (0, 0)),
        pl.BlockSpec((_H, _H), lambda i: (0, 0)),
        pl.BlockSpec((1, _H), lambda i: (0, 0)),
        pl.BlockSpec((_H, _H), lambda i: (0, 0)),
        pl.BlockSpec((1, _H), lambda i: (0, 0)),
    ],
    out_specs=[
        pl.BlockSpec((_RB, _H), lambda i: (i, 0)),
        pl.BlockSpec((8, _H), lambda i: (0, 0)),
    ],
    out_shape=[
        jax.ShapeDtypeStruct((_N, _H), jnp.float32),
        jax.ShapeDtypeStruct((8, _H), jnp.float32),
    ],
    compiler_params=pltpu.CompilerParams(dimension_semantics=("arbitrary",)),
)


def _aff_body(z_ref, a_ref, b_ref, o_ref):
    o_ref[...] = z_ref[...] * a_ref[...] + b_ref[...]


_aff_call = pl.pallas_call(
    _aff_body,
    grid=(_NRB,),
    in_specs=[
        pl.BlockSpec((_RB, _H), lambda i: (i, 0)),
        pl.BlockSpec((1, _H), lambda i: (0, 0)),
        pl.BlockSpec((1, _H), lambda i: (0, 0)),
    ],
    out_specs=pl.BlockSpec((_RB, _H), lambda i: (i, 0)),
    out_shape=jax.ShapeDtypeStruct((_N, _H), jnp.float32),
    compiler_params=pltpu.CompilerParams(dimension_semantics=("arbitrary",)),
)


def _pool_body(z_ref, bt_ref, a_ref, b_ref, w1_ref, b1_ref, w2r_ref, b2_ref,
               o_ref, accp, accc):
    i = pl.program_id(0)

    @pl.when(i == 0)
    def _():
        accp[...] = jnp.zeros_like(accp)
        accc[...] = jnp.zeros_like(accc)

    bt = bt_ref[0]
    gi = lax.broadcasted_iota(jnp.int32, (_G, _RB), 0)
    oh = (gi == bt).astype(jnp.float32)
    accp[...] += lax.dot_general(oh, z_ref[...], (((1,), (0,)), ((), ())),
                                 preferred_element_type=jnp.float32)
    accc[...] += jnp.broadcast_to(jnp.sum(oh, axis=1, keepdims=True), (_G, _H))

    @pl.when(i == pl.num_programs(0) - 1)
    def _():
        pooled = accp[...] / jnp.maximum(accc[...], 1.0)
        pooled = pooled * a_ref[...] + b_ref[...]
        r1 = jnp.maximum(
            jnp.dot(pooled, w1_ref[...], preferred_element_type=jnp.float32)
            + b1_ref[...], 0.0)
        o_ref[...] = jnp.sum(r1 * w2r_ref[...], axis=1, keepdims=True) + b2_ref[...]


_pool_call = pl.pallas_call(
    _pool_body,
    grid=(_NRB,),
    in_specs=[
        pl.BlockSpec((_RB, _H), lambda i: (i, 0)),
        pl.BlockSpec((1, 1, _RB), lambda i: (i, 0, 0)),
        pl.BlockSpec((1, _H), lambda i: (0, 0)),
        pl.BlockSpec((1, _H), lambda i: (0, 0)),
        pl.BlockSpec((_H, _H), lambda i: (0, 0)),
        pl.BlockSpec((1, _H), lambda i: (0, 0)),
        pl.BlockSpec((1, _H), lambda i: (0, 0)),
        pl.BlockSpec((1, 1), lambda i: (0, 0)),
    ],
    out_specs=pl.BlockSpec((_G, 1), lambda i: (0, 0)),
    out_shape=jax.ShapeDtypeStruct((_G, 1), jnp.float32),
    scratch_shapes=[
        pltpu.VMEM((_G, _H), jnp.float32),
        pltpu.VMEM((_G, _H), jnp.float32),
    ],
    compiler_params=pltpu.CompilerParams(dimension_semantics=("arbitrary",)),
)


def kernel(x, edge_index, batch, W1s, b1s, W2s, b2s, gammas, betas, eps_vec,
           lin1_W, lin1_b, lin2_W, lin2_b):
    src = edge_index[0]
    dst = edge_index[1]
    batch3 = batch.reshape(_NRB, 1, _RB)
    ones_row = jnp.ones((1, _H), jnp.float32)

    h = x
    out = None
    num_layers = W1s.shape[0]
    for l in range(num_layers):
        p0, p1 = _make_sc_segsum()(h, src, dst)
        scal = (1.0 + eps_vec[l]) * ones_row
        z2, st = _mlp_call(h, p0, p1, scal, W1s[l], b1s[l].reshape(1, _H),
                           W2s[l], b2s[l].reshape(1, _H))
        mu = st[0] / _N
        var = st[1] / _N - mu * mu
        a = gammas[l] * lax.rsqrt(var + 1e-5)
        bb = betas[l] - mu * a
        if l < num_layers - 1:
            h = _aff_call(z2, a.reshape(1, _H), bb.reshape(1, _H))
        else:
            out = _pool_call(z2, batch3, a.reshape(1, _H), bb.reshape(1, _H),
                             lin1_W, lin1_b.reshape(1, _H),
                             lin2_W.reshape(1, _H), lin2_b.reshape(1, 1))
    return out


# speedup vs baseline: 12.0203x; 2.6420x over previous
import functools

import jax
import jax.numpy as jnp
from jax import lax
from jax.experimental import pallas as pl
from jax.experimental.pallas import tpu as pltpu
from jax.experimental.pallas import tpu_sc as plsc

_N = 10000
_E = 320000
_H = 128
_G = 64
_NC = 2
_NS = 16
_NW = _NC * _NS
_EPW = _E // _NW
_CH = 80
_NCHUNK = _EPW // _CH
_NBUF = 3
_NIDX = 2 * _NBUF
_MAIN = (_NCHUNK // _NIDX) * _NIDX
_RPT = 624
_TAIL = _N - _NS * _RPT
_RB = 1000
_NRB = _N // _RB


@functools.cache
def _make_sc_segsum():
    mesh = plsc.VectorSubcoreMesh(core_axis_name="c", subcore_axis_name="s")
    out_t = (jax.ShapeDtypeStruct((_N, _H), jnp.float32),
             jax.ShapeDtypeStruct((_N, _H), jnp.float32))

    @functools.partial(
        pl.kernel, mesh=mesh, out_type=out_t,
        scratch_types=[
            pltpu.VMEM((_NIDX, _CH), jnp.int32),
            pltpu.VMEM((_NIDX, _CH), jnp.int32),
            pltpu.VMEM((_NBUF, _CH, _H), jnp.float32),
            pltpu.VMEM((8, _H), jnp.float32),
            pltpu.VMEM_SHARED((_N, _H), jnp.float32),
        ] + [pltpu.SemaphoreType.DMA] * (_NBUF + _NIDX))
    def seg(h_hbm, src_hbm, dst_hbm, out0, out1, sidx, didx, rows_v, zbuf,
            acc, *sems):
        gsems = sems[:_NBUF]
        isems = sems[_NBUF:]
        cid = lax.axis_index("c")
        sid = lax.axis_index("s")
        wid = cid * _NS + sid

        def issue_idx(ch, slot):
            e0 = wid * _EPW + ch * _CH
            pltpu.async_copy(src_hbm.at[pl.ds(e0, _CH)], sidx.at[slot],
                             isems[slot])
            pltpu.async_copy(dst_hbm.at[pl.ds(e0, _CH)], didx.at[slot],
                             isems[slot])

        def wait_idx(slot):
            pltpu.make_async_copy(src_hbm.at[pl.ds(0, _CH)], sidx.at[slot],
                                  isems[slot]).wait()
            pltpu.make_async_copy(dst_hbm.at[pl.ds(0, _CH)], didx.at[slot],
                                  isems[slot]).wait()

        def issue_gather(ch, slot, b):
            pltpu.async_copy(h_hbm.at[sidx.at[slot]], rows_v.at[b], gsems[b])

        def wait_gather(b):
            pltpu.make_async_copy(h_hbm.at[sidx.at[0]], rows_v.at[b],
                                  gsems[b]).wait()

        for s in range(_NIDX):
            issue_idx(s, s)
        for b in range(_NBUF):
            wait_idx(b)
            issue_gather(b, b, b)

        for i in range(8):
            for j in range(_H // 16):
                zbuf[i, pl.ds(j * 16, 16)] = jnp.zeros((16,), jnp.float32)
        base_r = sid * _RPT

        def zacc(j, c):
            pltpu.sync_copy(zbuf, acc.at[pl.ds(base_r + j * 8, 8)])
            return c
        lax.fori_loop(0, _RPT // 8, zacc, 0)

        @pl.when(sid == _NS - 1)
        def _():
            pltpu.sync_copy(zbuf.at[pl.ds(0, _TAIL)],
                            acc.at[pl.ds(_NS * _RPT, _TAIL)])
        plsc.subcore_barrier()

        def body(i, b, ib, guard):
            wait_gather(b)
            pltpu.sync_copy(rows_v.at[b], acc.at[didx.at[ib]], add=True)
            if guard:
                @pl.when(i + _NIDX < _NCHUNK)
                def _():
                    issue_idx(i + _NIDX, ib)

                @pl.when(i + _NBUF < _NCHUNK)
                def _():
                    wait_idx((ib + _NBUF) % _NIDX)
                    issue_gather(i + _NBUF, (ib + _NBUF) % _NIDX, b)
            else:
                issue_idx(i + _NIDX, ib)
                wait_idx((ib + _NBUF) % _NIDX)
                issue_gather(i + _NBUF, (ib + _NBUF) % _NIDX, b)

        def step(j, c):
            for bb in range(_NIDX):
                i = j * _NIDX + bb
                body(i, bb % _NBUF, bb, guard=False)
            return c
        lax.fori_loop(0, (_MAIN - _NIDX) // _NIDX, step, 0)
        for i in range(_MAIN - _NIDX, _NCHUNK):
            body(i, i % _NBUF, i % _NIDX, guard=True)
        plsc.subcore_barrier()

        @pl.when(cid == 0)
        def _():
            pltpu.sync_copy(acc.at[pl.ds(base_r, _RPT)], out0.at[pl.ds(base_r, _RPT)])

            @pl.when(sid == _NS - 1)
            def _():
                pltpu.sync_copy(acc.at[pl.ds(_NS * _RPT, _TAIL)],
                                out0.at[pl.ds(_NS * _RPT, _TAIL)])

        @pl.when(cid == 1)
        def _():
            pltpu.sync_copy(acc.at[pl.ds(base_r, _RPT)], out1.at[pl.ds(base_r, _RPT)])

            @pl.when(sid == _NS - 1)
            def _():
                pltpu.sync_copy(acc.at[pl.ds(_NS * _RPT, _TAIL)],
                                out1.at[pl.ds(_NS * _RPT, _TAIL)])

    return seg


def _mlp_body(h_ref, p0_ref, p1_ref, sc_ref, w1_ref, b1_ref, w2_ref, b2_ref,
              z2_ref, st_ref):
    z = h_ref[...] * sc_ref[...] + (p0_ref[...] + p1_ref[...])
    z1 = jnp.maximum(jnp.dot(z, w1_ref[...], preferred_element_type=jnp.float32)
                     + b1_ref[...], 0.0)
    z2 = jnp.maximum(jnp.dot(z1, w2_ref[...], preferred_element_type=jnp.float32)
                     + b2_ref[...], 0.0)
    z2_ref[...] = z2
    s = jnp.sum(z2, axis=0, keepdims=True)
    ss = jnp.sum(z2 * z2, axis=0, keepdims=True)
    upd = jnp.concatenate([s, ss, jnp.zeros((6, _H), jnp.float32)], axis=0)

    @pl.when(pl.program_id(0) == 0)
    def _():
        st_ref[...] = jnp.zeros_like(st_ref)

    st_ref[...] += upd


_mlp_call = pl.pallas_call(
    _mlp_body,
    grid=(_NRB,),
    in_specs=[
        pl.BlockSpec((_RB, _H), lambda i: (i, 0)),
        pl.BlockSpec((_RB, _H), lambda i: (i, 0)),
        pl.BlockSpec((_RB, _H), lambda i: (i, 0)),
        pl.BlockSpec((1, _H), lambda i: (0, 0)),
        pl.BlockSpec((_H, _H), lambda i: (0, 0)),
        pl.BlockSpec((1, _H), lambda i: (0, 0)),
        pl.BlockSpec((_H, _H), lambda i: (0, 0)),
        pl.BlockSpec((1, _H), lambda i: (0, 0)),
    ],
    out_specs=[
        pl.BlockSpec((_RB, _H), lambda i: (i, 0)),
        pl.BlockSpec((8, _H), lambda i: (0, 0)),
    ],
    out_shape=[
        jax.ShapeDtypeStruct((_N, _H), jnp.float32),
        jax.ShapeDtypeStruct((8, _H), jnp.float32),
    ],
    compiler_params=pltpu.CompilerParams(dimension_semantics=("arbitrary",)),
)


def _aff_body(z_ref, a_ref, b_ref, o_ref):
    o_ref[...] = z_ref[...] * a_ref[...] + b_ref[...]


_aff_call = pl.pallas_call(
    _aff_body,
    grid=(_NRB,),
    in_specs=[
        pl.BlockSpec((_RB, _H), lambda i: (i, 0)),
        pl.BlockSpec((1, _H), lambda i: (0, 0)),
        pl.BlockSpec((1, _H), lambda i: (0, 0)),
    ],
    out_specs=pl.BlockSpec((_RB, _H), lambda i: (i, 0)),
    out_shape=jax.ShapeDtypeStruct((_N, _H), jnp.float32),
    compiler_params=pltpu.CompilerParams(dimension_semantics=("arbitrary",)),
)


def _pool_body(z_ref, bt_ref, a_ref, b_ref, w1_ref, b1_ref, w2r_ref, b2_ref,
               o_ref, accp, accc):
    i = pl.program_id(0)

    @pl.when(i == 0)
    def _():
        accp[...] = jnp.zeros_like(accp)
        accc[...] = jnp.zeros_like(accc)

    bt = bt_ref[0]
    gi = lax.broadcasted_iota(jnp.int32, (_G, _RB), 0)
    oh = (gi == bt).astype(jnp.float32)
    accp[...] += lax.dot_general(oh, z_ref[...], (((1,), (0,)), ((), ())),
                                 preferred_element_type=jnp.float32)
    accc[...] += jnp.broadcast_to(jnp.sum(oh, axis=1, keepdims=True), (_G, _H))

    @pl.when(i == pl.num_programs(0) - 1)
    def _():
        pooled = accp[...] / jnp.maximum(accc[...], 1.0)
        pooled = pooled * a_ref[...] + b_ref[...]
        r1 = jnp.maximum(
            jnp.dot(pooled, w1_ref[...], preferred_element_type=jnp.float32)
            + b1_ref[...], 0.0)
        o_ref[...] = jnp.sum(r1 * w2r_ref[...], axis=1, keepdims=True) + b2_ref[...]


_pool_call = pl.pallas_call(
    _pool_body,
    grid=(_NRB,),
    in_specs=[
        pl.BlockSpec((_RB, _H), lambda i: (i, 0)),
        pl.BlockSpec((1, 1, _RB), lambda i: (i, 0, 0)),
        pl.BlockSpec((1, _H), lambda i: (0, 0)),
        pl.BlockSpec((1, _H), lambda i: (0, 0)),
        pl.BlockSpec((_H, _H), lambda i: (0, 0)),
        pl.BlockSpec((1, _H), lambda i: (0, 0)),
        pl.BlockSpec((1, _H), lambda i: (0, 0)),
        pl.BlockSpec((1, 1), lambda i: (0, 0)),
    ],
    out_specs=pl.BlockSpec((_G, 1), lambda i: (0, 0)),
    out_shape=jax.ShapeDtypeStruct((_G, 1), jnp.float32),
    scratch_shapes=[
        pltpu.VMEM((_G, _H), jnp.float32),
        pltpu.VMEM((_G, _H), jnp.float32),
    ],
    compiler_params=pltpu.CompilerParams(dimension_semantics=("arbitrary",)),
)


def kernel(x, edge_index, batch, W1s, b1s, W2s, b2s, gammas, betas, eps_vec,
           lin1_W, lin1_b, lin2_W, lin2_b):
    src = edge_index[0]
    dst = edge_index[1]
    batch3 = batch.reshape(_NRB, 1, _RB)
    ones_row = jnp.ones((1, _H), jnp.float32)

    h = x
    out = None
    num_layers = W1s.shape[0]
    for l in range(num_layers):
        p0, p1 = _make_sc_segsum()(h, src, dst)
        scal = (1.0 + eps_vec[l]) * ones_row
        z2, st = _mlp_call(h, p0, p1, scal, W1s[l], b1s[l].reshape(1, _H),
                           W2s[l], b2s[l].reshape(1, _H))
        mu = st[0] / _N
        var = st[1] / _N - mu * mu
        a = gammas[l] * lax.rsqrt(var + 1e-5)
        bb = betas[l] - mu * a
        if l < num_layers - 1:
            h = _aff_call(z2, a.reshape(1, _H), bb.reshape(1, _H))
        else:
            out = _pool_call(z2, batch3, a.reshape(1, _H), bb.reshape(1, _H),
                             lin1_W, lin1_b.reshape(1, _H),
                             lin2_W.reshape(1, _H), lin2_b.reshape(1, 1))
    return out
